# initial kernel scaffold (unmeasured)
import jax
import jax.numpy as jnp
from jax import lax
from jax.experimental import pallas as pl
from jax.experimental.pallas import tpu as pltpu

B, S, H, Dh, Dr = 4, 256, 32, 128, 64
D = 4096
DCS = 128
BS = B * S
SCALE = (Dh + Dr) ** -0.5
BF16 = jnp.bfloat16
F32 = jnp.float32


def _kv_body(x_ref, wdkv_ref, wuk_ref, wuv_ref, k_ref, v_ref,
             c_loc, c_rem, wuk_loc, wuk_rem, wuv_loc, wuv_rem,
             send_sems, recv_sems):
    my_x = lax.axis_index("x")
    my_y = lax.axis_index("y")
    my_z = lax.axis_index("z")
    partner = (my_x, my_y, 1 - my_z)

    x2 = x_ref[...].reshape(BS, D).astype(BF16)
    c_loc[...] = jnp.dot(
        x2, wdkv_ref[...].astype(BF16), preferred_element_type=F32
    ).astype(BF16)
    wuk_loc[...] = wuk_ref[...].astype(BF16)
    wuv_loc[...] = wuv_ref[...].astype(BF16)

    barrier = pltpu.get_barrier_semaphore()
    pl.semaphore_signal(
        barrier, inc=1, device_id=partner, device_id_type=pl.DeviceIdType.MESH
    )
    pl.semaphore_wait(barrier, 1)

    pairs = ((c_loc, c_rem), (wuk_loc, wuk_rem), (wuv_loc, wuv_rem))
    rdmas = []
    for i, (src, dst) in enumerate(pairs):
        r = pltpu.make_async_remote_copy(
            src_ref=src,
            dst_ref=dst,
            send_sem=send_sems.at[i],
            recv_sem=recv_sems.at[i],
            device_id=partner,
            device_id_type=pl.DeviceIdType.MESH,
        )
        r.start()
        rdmas.append(r)
    for r in rdmas:
        r.wait()

    k = jnp.dot(c_loc[...], wuk_loc[...], preferred_element_type=F32)
    k += jnp.dot(c_rem[...], wuk_rem[...], preferred_element_type=F32)
    k_ref[...] = k.astype(BF16)
    v = jnp.dot(c_loc[...], wuv_loc[...], preferred_element_type=F32)
    v += jnp.dot(c_rem[...], wuv_rem[...], preferred_element_type=F32)
    v_ref[...] = v.astype(BF16)


def _attn_body(x_ref, wq_ref, wqr_ref, wkr_ref, wo_ref, k_ref, v_ref,
               o_ref, kr_scr):
    h = pl.program_id(1)
    xb = x_ref[0].astype(BF16)

    @pl.when(h == 0)
    def _():
        kr_scr[...] = jnp.dot(
            xb, wkr_ref[...].astype(BF16), preferred_element_type=F32
        ).astype(BF16)

    q = jnp.dot(xb, wq_ref[...].astype(BF16), preferred_element_type=F32)
    qr = jnp.dot(xb, wqr_ref[...].astype(BF16), preferred_element_type=F32)
    s = jnp.dot(q.astype(BF16), k_ref[...].T, preferred_element_type=F32)
    s += jnp.dot(qr.astype(BF16), kr_scr[...].T, preferred_element_type=F32)
    s *= SCALE
    m = jnp.max(s, axis=-1, keepdims=True)
    p = jnp.exp(s - m)
    p = p / jnp.sum(p, axis=-1, keepdims=True)
    o = jnp.dot(p.astype(BF16), v_ref[...], preferred_element_type=F32)
    contrib = jnp.dot(
        o.astype(BF16), wo_ref[...].astype(BF16), preferred_element_type=F32
    )

    @pl.when(h == 0)
    def _():
        o_ref[0] = contrib

    @pl.when(h != 0)
    def _():
        o_ref[0] += contrib


def kernel(x, Wdkv, Wuk, Wuv, Wq, Wqr, Wkr, Wo):
    K, V = pl.pallas_call(
        _kv_body,
        out_shape=[jax.ShapeDtypeStruct((BS, H * Dh), BF16)] * 2,
        in_specs=[pl.BlockSpec(memory_space=pltpu.VMEM)] * 4,
        out_specs=[pl.BlockSpec(memory_space=pltpu.VMEM)] * 2,
        scratch_shapes=[
            pltpu.VMEM((BS, DCS), BF16),
            pltpu.VMEM((BS, DCS), BF16),
            pltpu.VMEM((DCS, H * Dh), BF16),
            pltpu.VMEM((DCS, H * Dh), BF16),
            pltpu.VMEM((DCS, H * Dh), BF16),
            pltpu.VMEM((DCS, H * Dh), BF16),
            pltpu.SemaphoreType.DMA((3,)),
            pltpu.SemaphoreType.DMA((3,)),
        ],
        compiler_params=pltpu.CompilerParams(collective_id=0),
    )(x, Wdkv, Wuk, Wuv)

    return pl.pallas_call(
        _attn_body,
        grid=(B, H),
        out_shape=jax.ShapeDtypeStruct((B, S, D), F32),
        in_specs=[
            pl.BlockSpec((1, S, D), lambda b, h: (b, 0, 0)),
            pl.BlockSpec((D, Dh), lambda b, h: (0, h)),
            pl.BlockSpec((D, Dr), lambda b, h: (0, h)),
            pl.BlockSpec((D, Dr), lambda b, h: (0, 0)),
            pl.BlockSpec((Dh, D), lambda b, h: (h, 0)),
            pl.BlockSpec((S, Dh), lambda b, h: (b, h)),
            pl.BlockSpec((S, Dh), lambda b, h: (b, h)),
        ],
        out_specs=pl.BlockSpec((1, S, D), lambda b, h: (b, 0, 0)),
        scratch_shapes=[pltpu.VMEM((S, Dr), BF16)],
        compiler_params=pltpu.CompilerParams(
            dimension_semantics=("arbitrary", "arbitrary")
        ),
    )(x, Wq, Wqr, Wkr, Wo, K, V)


# baseline (device time: 310421 ns/iter reference)
import jax
import jax.numpy as jnp
from jax import lax
from jax.experimental import pallas as pl
from jax.experimental.pallas import tpu as pltpu

B, S, H, Dh, Dr = 4, 256, 32, 128, 64
D = 4096
DCS = 128
BS = B * S
SCALE = (Dh + Dr) ** -0.5
BF16 = jnp.bfloat16
F32 = jnp.float32


def _kv_body(x_ref, wdkv_ref, wuk_ref, wuv_ref, k_ref, v_ref,
             c_loc, c_rem, wuk_loc, wuk_rem, wuv_loc, wuv_rem,
             send_sems, recv_sems):
    my_x = lax.axis_index("x")
    my_y = lax.axis_index("y")
    my_z = lax.axis_index("z")
    partner = (my_x, my_y, 1 - my_z)

    wdkv = wdkv_ref[...].astype(BF16)
    for b in range(B):
        xb = x_ref[b].astype(BF16)
        c_loc[pl.ds(b * S, S), :] = jnp.dot(
            xb, wdkv, preferred_element_type=F32
        ).astype(BF16)
    wuk_loc[...] = wuk_ref[...].astype(BF16)
    wuv_loc[...] = wuv_ref[...].astype(BF16)

    barrier = pltpu.get_barrier_semaphore()
    pl.semaphore_signal(
        barrier, inc=1, device_id=partner, device_id_type=pl.DeviceIdType.MESH
    )
    pl.semaphore_wait(barrier, 1)

    pairs = ((c_loc, c_rem), (wuk_loc, wuk_rem), (wuv_loc, wuv_rem))
    rdmas = []
    for i, (src, dst) in enumerate(pairs):
        r = pltpu.make_async_remote_copy(
            src_ref=src,
            dst_ref=dst,
            send_sem=send_sems.at[i],
            recv_sem=recv_sems.at[i],
            device_id=partner,
            device_id_type=pl.DeviceIdType.MESH,
        )
        r.start()
        rdmas.append(r)
    for r in rdmas:
        r.wait()

    for b in range(B):
        rows = pl.ds(b * S, S)
        cl = c_loc[rows, :]
        cr = c_rem[rows, :]
        k = jnp.dot(cl, wuk_loc[...], preferred_element_type=F32)
        k += jnp.dot(cr, wuk_rem[...], preferred_element_type=F32)
        k_ref[rows, :] = k.astype(BF16)
        v = jnp.dot(cl, wuv_loc[...], preferred_element_type=F32)
        v += jnp.dot(cr, wuv_rem[...], preferred_element_type=F32)
        v_ref[rows, :] = v.astype(BF16)


HPB = 4


def _attn_body(x_ref, wq_ref, wqr_ref, wkr_ref, wo_ref, k_ref, v_ref,
               o_ref, kr_scr):
    g = pl.program_id(1)
    xb = x_ref[0].astype(BF16)

    @pl.when(g == 0)
    def _():
        kr_scr[...] = jnp.dot(
            xb, wkr_ref[...].astype(BF16), preferred_element_type=F32
        ).astype(BF16)

    q = jnp.dot(xb, wq_ref[...].astype(BF16), preferred_element_type=F32
                ).astype(BF16)
    qr = jnp.dot(xb, wqr_ref[...].astype(BF16), preferred_element_type=F32
                 ).astype(BF16)
    kr_t = kr_scr[...].T

    outs = []
    for hh in range(HPB):
        q_h = q[:, hh * Dh:(hh + 1) * Dh]
        qr_h = qr[:, hh * Dr:(hh + 1) * Dr]
        k_h = k_ref[:, hh * Dh:(hh + 1) * Dh]
        v_h = v_ref[:, hh * Dh:(hh + 1) * Dh]
        s = jnp.dot(q_h, k_h.T, preferred_element_type=F32)
        s += jnp.dot(qr_h, kr_t, preferred_element_type=F32)
        s *= SCALE
        m = jnp.max(s, axis=-1, keepdims=True)
        p = jnp.exp(s - m)
        p = p / jnp.sum(p, axis=-1, keepdims=True)
        outs.append(jnp.dot(p.astype(BF16), v_h, preferred_element_type=F32))
    o = jnp.concatenate(outs, axis=-1).astype(BF16)
    contrib = jnp.dot(
        o, wo_ref[...].astype(BF16), preferred_element_type=F32
    )

    @pl.when(g == 0)
    def _():
        o_ref[0] = contrib

    @pl.when(g != 0)
    def _():
        o_ref[0] += contrib


def kernel(x, Wdkv, Wuk, Wuv, Wq, Wqr, Wkr, Wo):
    K, V = pl.pallas_call(
        _kv_body,
        out_shape=[jax.ShapeDtypeStruct((BS, H * Dh), BF16)] * 2,
        in_specs=[pl.BlockSpec(memory_space=pltpu.VMEM)] * 4,
        out_specs=[pl.BlockSpec(memory_space=pltpu.VMEM)] * 2,
        scratch_shapes=[
            pltpu.VMEM((BS, DCS), BF16),
            pltpu.VMEM((BS, DCS), BF16),
            pltpu.VMEM((DCS, H * Dh), BF16),
            pltpu.VMEM((DCS, H * Dh), BF16),
            pltpu.VMEM((DCS, H * Dh), BF16),
            pltpu.VMEM((DCS, H * Dh), BF16),
            pltpu.SemaphoreType.DMA((3,)),
            pltpu.SemaphoreType.DMA((3,)),
        ],
        compiler_params=pltpu.CompilerParams(
            collective_id=0, vmem_limit_bytes=100 * 1024 * 1024
        ),
    )(x, Wdkv, Wuk, Wuv)

    return pl.pallas_call(
        _attn_body,
        grid=(B, H // HPB),
        out_shape=jax.ShapeDtypeStruct((B, S, D), F32),
        in_specs=[
            pl.BlockSpec((1, S, D), lambda b, g: (b, 0, 0)),
            pl.BlockSpec((D, HPB * Dh), lambda b, g: (0, g)),
            pl.BlockSpec((D, HPB * Dr), lambda b, g: (0, g)),
            pl.BlockSpec((D, Dr), lambda b, g: (0, 0)),
            pl.BlockSpec((HPB * Dh, D), lambda b, g: (g, 0)),
            pl.BlockSpec((S, HPB * Dh), lambda b, g: (b, g)),
            pl.BlockSpec((S, HPB * Dh), lambda b, g: (b, g)),
        ],
        out_specs=pl.BlockSpec((1, S, D), lambda b, g: (b, 0, 0)),
        scratch_shapes=[pltpu.VMEM((S, Dr), BF16)],
        compiler_params=pltpu.CompilerParams(
            dimension_semantics=("arbitrary", "arbitrary"),
            vmem_limit_bytes=100 * 1024 * 1024,
        ),
    )(x, Wq, Wqr, Wkr, Wo, K, V)


# device time: 187644 ns/iter; 1.6543x vs baseline; 1.6543x over previous
import jax
import jax.numpy as jnp
from jax import lax
from jax.experimental import pallas as pl
from jax.experimental.pallas import tpu as pltpu

B, S, H, Dh, Dr = 4, 256, 32, 128, 64
D = 4096
DCS = 128
SCALE = (Dh + Dr) ** -0.5
BF16 = jnp.bfloat16
F32 = jnp.float32
HPB = 4
G = H // HPB

_MESH = pl.DeviceIdType.MESH


def _kv_body(x_ref, wdkv_ref, wuk_ref, wuv_ref, xq_ref, k_ref, v_ref,
             c_loc, c_rem, wuk_loc, wuk_rem, wuv_loc, wuv_rem,
             send_sems, recv_sems):
    my_x = lax.axis_index("x")
    my_y = lax.axis_index("y")
    my_z = lax.axis_index("z")
    q = 2 * my_x + my_y
    partner = (my_x, my_y, 1 - my_z)

    xq = x_ref[pl.ds(q, 1), :, :].reshape(S, D).astype(BF16)
    xq_ref[...] = xq
    c_loc[...] = jnp.dot(
        xq, wdkv_ref[...].astype(BF16), preferred_element_type=F32
    ).astype(BF16)
    wuk_loc[...] = wuk_ref[...].astype(BF16)
    wuv_loc[...] = wuv_ref[...].astype(BF16)

    barrier = pltpu.get_barrier_semaphore()
    pl.semaphore_signal(barrier, inc=1, device_id=partner, device_id_type=_MESH)
    pl.semaphore_wait(barrier, 1)

    pairs = ((c_loc, c_rem), (wuk_loc, wuk_rem), (wuv_loc, wuv_rem))
    rdmas = []
    for i, (src, dst) in enumerate(pairs):
        r = pltpu.make_async_remote_copy(
            src_ref=src, dst_ref=dst,
            send_sem=send_sems.at[i], recv_sem=recv_sems.at[i],
            device_id=partner, device_id_type=_MESH,
        )
        r.start()
        rdmas.append(r)
    for r in rdmas:
        r.wait()

    k = jnp.dot(c_loc[...], wuk_loc[...], preferred_element_type=F32)
    k += jnp.dot(c_rem[...], wuk_rem[...], preferred_element_type=F32)
    k_ref[...] = k.astype(BF16)
    v = jnp.dot(c_loc[...], wuv_loc[...], preferred_element_type=F32)
    v += jnp.dot(c_rem[...], wuv_rem[...], preferred_element_type=F32)
    v_ref[...] = v.astype(BF16)


def _attn_body(xq_ref, wq_ref, wqr_ref, wkr_ref, wo_ref, k_ref, v_ref,
               o_ref, acc, kr_scr):
    g = pl.program_id(0)
    xb = xq_ref[...]

    @pl.when(g == 0)
    def _():
        kr_scr[...] = jnp.dot(
            xb, wkr_ref[...].astype(BF16), preferred_element_type=F32
        ).astype(BF16)

    qp = jnp.dot(xb, wq_ref[...].astype(BF16), preferred_element_type=F32
                 ).astype(BF16)
    qr = jnp.dot(xb, wqr_ref[...].astype(BF16), preferred_element_type=F32
                 ).astype(BF16)
    kr_t = kr_scr[...].T

    outs = []
    for hh in range(HPB):
        q_h = qp[:, hh * Dh:(hh + 1) * Dh]
        qr_h = qr[:, hh * Dr:(hh + 1) * Dr]
        k_h = k_ref[:, hh * Dh:(hh + 1) * Dh]
        v_h = v_ref[:, hh * Dh:(hh + 1) * Dh]
        s = jnp.dot(q_h, k_h.T, preferred_element_type=F32)
        s += jnp.dot(qr_h, kr_t, preferred_element_type=F32)
        s *= SCALE
        m = jnp.max(s, axis=-1, keepdims=True)
        p = jnp.exp(s - m)
        p = p / jnp.sum(p, axis=-1, keepdims=True)
        outs.append(jnp.dot(p.astype(BF16), v_h, preferred_element_type=F32))
    o = jnp.concatenate(outs, axis=-1).astype(BF16)
    contrib = jnp.dot(o, wo_ref[...].astype(BF16), preferred_element_type=F32)

    @pl.when(g == 0)
    def _():
        acc[...] = contrib

    @pl.when(g != 0)
    def _():
        acc[...] += contrib

    @pl.when(g == G - 1)
    def _():
        o_ref[...] = acc[...].astype(BF16)


def _gather_body(mine_ref, out_ref, buf_x, buf_y, buf_d,
                 send_sems, recv_sems):
    my_x = lax.axis_index("x")
    my_y = lax.axis_index("y")
    my_z = lax.axis_index("z")
    q = 2 * my_x + my_y
    q_xn = 2 * (1 - my_x) + my_y
    q_yn = 2 * my_x + (1 - my_y)
    q_dg = 2 * (1 - my_x) + (1 - my_y)
    xn = (1 - my_x, my_y, my_z)
    yn = (my_x, 1 - my_y, my_z)

    def store(qi, val):
        out_ref[pl.ds(qi, 1), :, :] = val.astype(F32).reshape(1, S, D)

    store(q, mine_ref[...])

    barrier = pltpu.get_barrier_semaphore()
    pl.semaphore_signal(barrier, inc=1, device_id=xn, device_id_type=_MESH)
    pl.semaphore_signal(barrier, inc=1, device_id=yn, device_id_type=_MESH)
    pl.semaphore_wait(barrier, 2)

    r1 = pltpu.make_async_remote_copy(
        src_ref=mine_ref, dst_ref=buf_x,
        send_sem=send_sems.at[0], recv_sem=recv_sems.at[0],
        device_id=xn, device_id_type=_MESH,
    )
    r1.start()
    r2 = pltpu.make_async_remote_copy(
        src_ref=mine_ref, dst_ref=buf_y,
        send_sem=send_sems.at[1], recv_sem=recv_sems.at[1],
        device_id=yn, device_id_type=_MESH,
    )
    r2.start()

    r1.wait_recv()
    r3 = pltpu.make_async_remote_copy(
        src_ref=buf_x, dst_ref=buf_d,
        send_sem=send_sems.at[2], recv_sem=recv_sems.at[2],
        device_id=yn, device_id_type=_MESH,
    )
    r3.start()
    store(q_xn, buf_x[...])

    r2.wait_recv()
    store(q_yn, buf_y[...])
    r3.wait_recv()
    store(q_dg, buf_d[...])

    r1.wait_send()
    r2.wait_send()
    r3.wait_send()


def kernel(x, Wdkv, Wuk, Wuv, Wq, Wqr, Wkr, Wo):
    xq, K, V = pl.pallas_call(
        _kv_body,
        out_shape=[jax.ShapeDtypeStruct((S, D), BF16)] * 3,
        in_specs=[pl.BlockSpec(memory_space=pltpu.VMEM)] * 4,
        out_specs=[pl.BlockSpec(memory_space=pltpu.VMEM)] * 3,
        scratch_shapes=[
            pltpu.VMEM((S, DCS), BF16),
            pltpu.VMEM((S, DCS), BF16),
            pltpu.VMEM((DCS, H * Dh), BF16),
            pltpu.VMEM((DCS, H * Dh), BF16),
            pltpu.VMEM((DCS, H * Dh), BF16),
            pltpu.VMEM((DCS, H * Dh), BF16),
            pltpu.SemaphoreType.DMA((3,)),
            pltpu.SemaphoreType.DMA((3,)),
        ],
        compiler_params=pltpu.CompilerParams(
            collective_id=0, vmem_limit_bytes=100 * 1024 * 1024
        ),
    )(x, Wdkv, Wuk, Wuv)

    mine = pl.pallas_call(
        _attn_body,
        grid=(G,),
        out_shape=jax.ShapeDtypeStruct((S, D), BF16),
        in_specs=[
            pl.BlockSpec((S, D), lambda g: (0, 0)),
            pl.BlockSpec((D, HPB * Dh), lambda g: (0, g)),
            pl.BlockSpec((D, HPB * Dr), lambda g: (0, g)),
            pl.BlockSpec((D, Dr), lambda g: (0, 0)),
            pl.BlockSpec((HPB * Dh, D), lambda g: (g, 0)),
            pl.BlockSpec((S, HPB * Dh), lambda g: (0, g)),
            pl.BlockSpec((S, HPB * Dh), lambda g: (0, g)),
        ],
        out_specs=pl.BlockSpec((S, D), lambda g: (0, 0)),
        scratch_shapes=[
            pltpu.VMEM((S, D), F32),
            pltpu.VMEM((S, Dr), BF16),
        ],
        compiler_params=pltpu.CompilerParams(
            dimension_semantics=("arbitrary",),
            vmem_limit_bytes=100 * 1024 * 1024,
        ),
    )(xq, Wq, Wqr, Wkr, Wo, K, V)

    return pl.pallas_call(
        _gather_body,
        out_shape=jax.ShapeDtypeStruct((B, S, D), F32),
        in_specs=[pl.BlockSpec(memory_space=pltpu.VMEM)],
        out_specs=pl.BlockSpec(memory_space=pltpu.VMEM),
        scratch_shapes=[
            pltpu.VMEM((S, D), BF16),
            pltpu.VMEM((S, D), BF16),
            pltpu.VMEM((S, D), BF16),
            pltpu.SemaphoreType.DMA((3,)),
            pltpu.SemaphoreType.DMA((3,)),
        ],
        compiler_params=pltpu.CompilerParams(
            collective_id=1, vmem_limit_bytes=100 * 1024 * 1024
        ),
    )(mine)


# device time: 153924 ns/iter; 2.0167x vs baseline; 1.2191x over previous
import jax
import jax.numpy as jnp
from jax import lax
from jax.experimental import pallas as pl
from jax.experimental.pallas import tpu as pltpu

B, S, H, Dh, Dr = 4, 256, 32, 128, 64
D = 4096
D2 = D // 2
DCS = 128
SCALE = (Dh + Dr) ** -0.5
BF16 = jnp.bfloat16
F32 = jnp.float32
HPB = 2
G = H // HPB

_MESH = pl.DeviceIdType.MESH


def _z_rdmas(c_loc, c_rem, wuk_loc, wuk_rem, wuv_loc, wuv_rem,
             zsend, zrecv, zp):
    pairs = ((c_loc, c_rem), (wuk_loc, wuk_rem), (wuv_loc, wuv_rem))
    return [
        pltpu.make_async_remote_copy(
            src_ref=src, dst_ref=dst,
            send_sem=zsend.at[i], recv_sem=zrecv.at[i],
            device_id=zp, device_id_type=_MESH,
        )
        for i, (src, dst) in enumerate(pairs)
    ]


def _main_body(x_hbm, wdkv_ref, wuk_ref, wuv_ref, wkr_ref, wq_ref, wqr_ref,
               wo_ref, out_ref,
               xq32, xb, c_loc, c_rem, wuk_loc, wuk_rem, wuv_loc, wuv_rem,
               kbuf, vbuf, kr_scr, qall, qrall, acc,
               copy_sem, zsend, zrecv):
    t = pl.program_id(0)
    my_x = lax.axis_index("x")
    my_y = lax.axis_index("y")
    my_z = lax.axis_index("z")
    q = 2 * my_x + my_y
    zp = (my_x, my_y, 1 - my_z)

    @pl.when(t == 0)
    def _():
        cp = pltpu.make_async_copy(x_hbm.at[q], xq32, copy_sem)
        cp.start()
        cp.wait()
        xb[...] = xq32[...].astype(BF16)
        c_loc[...] = jnp.dot(
            xb[...], wdkv_ref[...].astype(BF16), preferred_element_type=F32
        ).astype(BF16)
        wuk_loc[...] = wuk_ref[...].astype(BF16)
        wuv_loc[...] = wuv_ref[...].astype(BF16)

        barrier = pltpu.get_barrier_semaphore()
        pl.semaphore_signal(
            barrier, inc=1, device_id=zp, device_id_type=_MESH
        )
        pl.semaphore_wait(barrier, 1)
        for r in _z_rdmas(c_loc, c_rem, wuk_loc, wuk_rem, wuv_loc, wuv_rem,
                          zsend, zrecv, zp):
            r.start()

        kr_scr[...] = jnp.dot(
            xb[...], wkr_ref[...].astype(BF16), preferred_element_type=F32
        ).astype(BF16)

    @pl.when(t < G)
    def _():
        xq = xb[...]
        qall[:, pl.ds(t * (HPB * Dh), HPB * Dh)] = jnp.dot(
            xq, wq_ref[...].astype(BF16), preferred_element_type=F32
        ).astype(BF16)
        qrall[:, pl.ds(t * (HPB * Dr), HPB * Dr)] = jnp.dot(
            xq, wqr_ref[...].astype(BF16), preferred_element_type=F32
        ).astype(BF16)

    @pl.when(t == G)
    def _():
        for r in _z_rdmas(c_loc, c_rem, wuk_loc, wuk_rem, wuv_loc, wuv_rem,
                          zsend, zrecv, zp):
            r.wait()
        k = jnp.dot(c_loc[...], wuk_loc[...], preferred_element_type=F32)
        k += jnp.dot(c_rem[...], wuk_rem[...], preferred_element_type=F32)
        kbuf[...] = k.astype(BF16)
        v = jnp.dot(c_loc[...], wuv_loc[...], preferred_element_type=F32)
        v += jnp.dot(c_rem[...], wuv_rem[...], preferred_element_type=F32)
        vbuf[...] = v.astype(BF16)

    @pl.when(t >= G)
    def _():
        g = t - G
        kr_t = kr_scr[...].T
        qr_g = qrall[:, pl.ds(g * (HPB * Dr), HPB * Dr)]
        outs = []
        for hh in range(HPB):
            col = pl.ds(g * (HPB * Dh) + hh * Dh, Dh)
            q_h = qall[:, col]
            qr_h = qr_g[:, hh * Dr:(hh + 1) * Dr]
            k_h = kbuf[:, col]
            v_h = vbuf[:, col]
            s = jnp.dot(q_h, k_h.T, preferred_element_type=F32)
            s += jnp.dot(qr_h, kr_t, preferred_element_type=F32)
            s *= SCALE
            m = jnp.max(s, axis=-1, keepdims=True)
            p = jnp.exp(s - m)
            p = p / jnp.sum(p, axis=-1, keepdims=True)
            outs.append(
                jnp.dot(p.astype(BF16), v_h, preferred_element_type=F32)
            )
        o = jnp.concatenate(outs, axis=-1).astype(BF16)
        contrib = jnp.dot(
            o, wo_ref[...].astype(BF16), preferred_element_type=F32
        )

        @pl.when(t == G)
        def _():
            acc[...] = contrib

        @pl.when(t != G)
        def _():
            acc[...] += contrib

        @pl.when(t == 2 * G - 1)
        def _():
            out_ref[...] = acc[...].astype(BF16)


def _gather_body(mine_ref, out_ref, buf_x, buf_y, buf_d, gsend, grecv):
    my_x = lax.axis_index("x")
    my_y = lax.axis_index("y")
    my_z = lax.axis_index("z")
    q = 2 * my_x + my_y
    q_xn = 2 * (1 - my_x) + my_y
    q_yn = 2 * my_x + (1 - my_y)
    q_dg = 2 * (1 - my_x) + (1 - my_y)
    xn = (1 - my_x, my_y, my_z)
    yn = (my_x, 1 - my_y, my_z)

    def store(qi, val):
        out_ref[pl.ds(qi, 1), :, :] = val.astype(F32).reshape(1, S, D)

    store(q, mine_ref[...])

    barrier = pltpu.get_barrier_semaphore()
    pl.semaphore_signal(barrier, inc=1, device_id=xn, device_id_type=_MESH)
    pl.semaphore_signal(barrier, inc=1, device_id=yn, device_id_type=_MESH)
    pl.semaphore_wait(barrier, 2)

    r1 = pltpu.make_async_remote_copy(
        src_ref=mine_ref, dst_ref=buf_x,
        send_sem=gsend.at[0], recv_sem=grecv.at[0],
        device_id=xn, device_id_type=_MESH,
    )
    r1.start()
    r2 = pltpu.make_async_remote_copy(
        src_ref=mine_ref, dst_ref=buf_y,
        send_sem=gsend.at[1], recv_sem=grecv.at[1],
        device_id=yn, device_id_type=_MESH,
    )
    r2.start()

    r1.wait_recv()
    rf_y = pltpu.make_async_remote_copy(
        src_ref=buf_x.at[:, pl.ds(0, D2)],
        dst_ref=buf_d.at[:, pl.ds(0, D2)],
        send_sem=gsend.at[2], recv_sem=grecv.at[2],
        device_id=yn, device_id_type=_MESH,
    )
    rf_y.start()
    store(q_xn, buf_x[...])

    r2.wait_recv()
    rf_x = pltpu.make_async_remote_copy(
        src_ref=buf_y.at[:, pl.ds(D2, D2)],
        dst_ref=buf_d.at[:, pl.ds(D2, D2)],
        send_sem=gsend.at[3], recv_sem=grecv.at[3],
        device_id=xn, device_id_type=_MESH,
    )
    rf_x.start()
    store(q_yn, buf_y[...])

    rf_y.wait_recv()
    rf_x.wait_recv()
    store(q_dg, buf_d[...])

    r1.wait_send()
    r2.wait_send()
    rf_y.wait_send()
    rf_x.wait_send()


def kernel(x, Wdkv, Wuk, Wuv, Wq, Wqr, Wkr, Wo):
    mine = pl.pallas_call(
        _main_body,
        grid=(2 * G,),
        out_shape=jax.ShapeDtypeStruct((S, D), BF16),
        in_specs=[
            pl.BlockSpec(memory_space=pl.ANY),
            pl.BlockSpec(memory_space=pltpu.VMEM),
            pl.BlockSpec(memory_space=pltpu.VMEM),
            pl.BlockSpec(memory_space=pltpu.VMEM),
            pl.BlockSpec(memory_space=pltpu.VMEM),
            pl.BlockSpec((D, HPB * Dh),
                         lambda t: (0, jnp.minimum(t, G - 1))),
            pl.BlockSpec((D, HPB * Dr),
                         lambda t: (0, jnp.minimum(t, G - 1))),
            pl.BlockSpec((HPB * Dh, D),
                         lambda t: (jnp.maximum(t - G, 0), 0)),
        ],
        out_specs=pl.BlockSpec(memory_space=pltpu.VMEM),
        scratch_shapes=[
            pltpu.VMEM((S, D), F32),
            pltpu.VMEM((S, D), BF16),
            pltpu.VMEM((S, DCS), BF16),
            pltpu.VMEM((S, DCS), BF16),
            pltpu.VMEM((DCS, H * Dh), BF16),
            pltpu.VMEM((DCS, H * Dh), BF16),
            pltpu.VMEM((DCS, H * Dh), BF16),
            pltpu.VMEM((DCS, H * Dh), BF16),
            pltpu.VMEM((S, H * Dh), BF16),
            pltpu.VMEM((S, H * Dh), BF16),
            pltpu.VMEM((S, Dr), BF16),
            pltpu.VMEM((S, H * Dh), BF16),
            pltpu.VMEM((S, H * Dr), BF16),
            pltpu.VMEM((S, D), F32),
            pltpu.SemaphoreType.DMA,
            pltpu.SemaphoreType.DMA((3,)),
            pltpu.SemaphoreType.DMA((3,)),
        ],
        compiler_params=pltpu.CompilerParams(
            dimension_semantics=("arbitrary",),
            collective_id=0,
            vmem_limit_bytes=62 * 1024 * 1024,
        ),
    )(x, Wdkv, Wuk, Wuv, Wkr, Wq, Wqr, Wo)

    return pl.pallas_call(
        _gather_body,
        out_shape=jax.ShapeDtypeStruct((B, S, D), F32),
        in_specs=[pl.BlockSpec(memory_space=pltpu.VMEM)],
        out_specs=pl.BlockSpec(memory_space=pltpu.VMEM),
        scratch_shapes=[
            pltpu.VMEM((S, D), BF16),
            pltpu.VMEM((S, D), BF16),
            pltpu.VMEM((S, D), BF16),
            pltpu.SemaphoreType.DMA((4,)),
            pltpu.SemaphoreType.DMA((4,)),
        ],
        compiler_params=pltpu.CompilerParams(
            collective_id=1, vmem_limit_bytes=62 * 1024 * 1024
        ),
    )(mine)


# device time: 128365 ns/iter; 2.4183x vs baseline; 1.1991x over previous
import jax
import jax.numpy as jnp
from jax import lax
from jax.experimental import pallas as pl
from jax.experimental.pallas import tpu as pltpu

B, S, H, Dh, Dr = 4, 256, 32, 128, 64
D = 4096
DCS = 128
SCALE = (Dh + Dr) ** -0.5
BF16 = jnp.bfloat16
F32 = jnp.float32
HPB = 2
G = H // HPB
NJ = 16
SW = D // NJ
SW2 = SW // 2

_MESH = pl.DeviceIdType.MESH


def _z_rdmas(c_loc, c_rem, wuk_loc, wuk_rem, wuv_loc, wuv_rem,
             zsend, zrecv, zp):
    pairs = ((c_loc, c_rem), (wuk_loc, wuk_rem), (wuv_loc, wuv_rem))
    return [
        pltpu.make_async_remote_copy(
            src_ref=src, dst_ref=dst,
            send_sem=zsend.at[i], recv_sem=zrecv.at[i],
            device_id=zp, device_id_type=_MESH,
        )
        for i, (src, dst) in enumerate(pairs)
    ]


def _main_body(x_hbm, wdkv_ref, wuk_ref, wuv_ref, wkr_ref, wo_hbm,
               wq_ref, wqr_ref, out_ref,
               xq32, xb, c_loc, c_rem, wuk_loc, wuk_rem, wuv_loc, wuv_rem,
               kbuf, vbuf, kr_scr, qall, qrall, o_all, wo_stage,
               copy_sem, wo_sems, zsend, zrecv, gsend, grecv):
    t = pl.program_id(0)
    my_x = lax.axis_index("x")
    my_y = lax.axis_index("y")
    my_z = lax.axis_index("z")
    q = 2 * my_x + my_y
    q_xn = 2 * (1 - my_x) + my_y
    q_yn = 2 * my_x + (1 - my_y)
    q_dg = 2 * (1 - my_x) + (1 - my_y)
    zp = (my_x, my_y, 1 - my_z)
    xn = (1 - my_x, my_y, my_z)
    yn = (my_x, 1 - my_y, my_z)

    @pl.when(t == 0)
    def _():
        cp = pltpu.make_async_copy(x_hbm.at[q], xq32, copy_sem)
        cp.start()
        cp.wait()
        xb[...] = xq32[...].astype(BF16)
        c_loc[...] = jnp.dot(
            xb[...], wdkv_ref[...].astype(BF16), preferred_element_type=F32
        ).astype(BF16)
        wuk_loc[...] = wuk_ref[...].astype(BF16)
        wuv_loc[...] = wuv_ref[...].astype(BF16)

        barrier = pltpu.get_barrier_semaphore()
        for nbr in (zp, xn, yn):
            pl.semaphore_signal(
                barrier, inc=1, device_id=nbr, device_id_type=_MESH
            )
        pl.semaphore_wait(barrier, 3)

        for r in _z_rdmas(c_loc, c_rem, wuk_loc, wuk_rem, wuv_loc, wuv_rem,
                          zsend, zrecv, zp):
            r.start()

        kr_scr[...] = jnp.dot(
            xb[...], wkr_ref[...].astype(BF16), preferred_element_type=F32
        ).astype(BF16)

    @pl.when(t < G)
    def _():
        xq = xb[...]
        qall[:, pl.ds(t * (HPB * Dh), HPB * Dh)] = jnp.dot(
            xq, wq_ref[...].astype(BF16), preferred_element_type=F32
        ).astype(BF16)
        qrall[:, pl.ds(t * (HPB * Dr), HPB * Dr)] = jnp.dot(
            xq, wqr_ref[...].astype(BF16), preferred_element_type=F32
        ).astype(BF16)

    @pl.when(t == G)
    def _():
        for r in _z_rdmas(c_loc, c_rem, wuk_loc, wuk_rem, wuv_loc, wuv_rem,
                          zsend, zrecv, zp):
            r.wait()
        for half in range(2):
            cols = pl.ds(half * (H * Dh // 2), H * Dh // 2)
            k = jnp.dot(c_loc[...], wuk_loc[:, cols],
                        preferred_element_type=F32)
            k += jnp.dot(c_rem[...], wuk_rem[:, cols],
                         preferred_element_type=F32)
            kbuf[:, cols] = k.astype(BF16)
            v = jnp.dot(c_loc[...], wuv_loc[:, cols],
                        preferred_element_type=F32)
            v += jnp.dot(c_rem[...], wuv_rem[:, cols],
                         preferred_element_type=F32)
            vbuf[:, cols] = v.astype(BF16)

    @pl.when(t >= G)
    def _():
        g = t - G
        kr_t = kr_scr[...].T
        qr_g = qrall[:, pl.ds(g * (HPB * Dr), HPB * Dr)]
        outs = []
        for hh in range(HPB):
            col = pl.ds(g * (HPB * Dh) + hh * Dh, Dh)
            q_h = qall[:, col]
            qr_h = qr_g[:, hh * Dr:(hh + 1) * Dr]
            k_h = kbuf[:, col]
            v_h = vbuf[:, col]
            s = jnp.dot(q_h, k_h.T, preferred_element_type=F32)
            s += jnp.dot(qr_h, kr_t, preferred_element_type=F32)
            s *= SCALE
            m = jnp.max(s, axis=-1, keepdims=True)
            p = jnp.exp(s - m)
            p = p / jnp.sum(p, axis=-1, keepdims=True)
            outs.append(
                jnp.dot(p.astype(BF16), v_h, preferred_element_type=F32)
            )
        o_all[:, pl.ds(g * (HPB * Dh), HPB * Dh)] = (
            jnp.concatenate(outs, axis=-1).astype(BF16)
        )

    @pl.when(t == 2 * G - 1)
    def _():
        def jcols(j, half=None):
            if half is None:
                return pl.ds(j * SW, SW)
            return pl.ds(j * SW + half * SW2, SW2)

        def stripe_rdmas(j):
            r1 = pltpu.make_async_remote_copy(
                src_ref=out_ref.at[q, :, jcols(j)],
                dst_ref=out_ref.at[q, :, jcols(j)],
                send_sem=gsend.at[0, j], recv_sem=grecv.at[0, j],
                device_id=xn, device_id_type=_MESH,
            )
            r2 = pltpu.make_async_remote_copy(
                src_ref=out_ref.at[q, :, jcols(j)],
                dst_ref=out_ref.at[q, :, jcols(j)],
                send_sem=gsend.at[1, j], recv_sem=grecv.at[1, j],
                device_id=yn, device_id_type=_MESH,
            )
            return r1, r2

        def forward_rdmas(j):
            fy = pltpu.make_async_remote_copy(
                src_ref=out_ref.at[q_xn, :, jcols(j, 0)],
                dst_ref=out_ref.at[q_xn, :, jcols(j, 0)],
                send_sem=gsend.at[2, j], recv_sem=grecv.at[2, j],
                device_id=yn, device_id_type=_MESH,
            )
            fx = pltpu.make_async_remote_copy(
                src_ref=out_ref.at[q_yn, :, jcols(j, 1)],
                dst_ref=out_ref.at[q_yn, :, jcols(j, 1)],
                send_sem=gsend.at[3, j], recv_sem=grecv.at[3, j],
                device_id=xn, device_id_type=_MESH,
            )
            return fy, fx

        ld0 = pltpu.make_async_copy(
            wo_hbm.at[:, jcols(0)], wo_stage.at[0], wo_sems.at[0]
        )
        ld0.start()
        for j in range(NJ):
            sl = j % 2
            pltpu.make_async_copy(
                wo_hbm.at[:, jcols(j)], wo_stage.at[sl], wo_sems.at[sl]
            ).wait()
            if j + 1 < NJ:
                pltpu.make_async_copy(
                    wo_hbm.at[:, jcols(j + 1)], wo_stage.at[1 - sl],
                    wo_sems.at[1 - sl],
                ).start()
            oblk = jnp.dot(
                o_all[...], wo_stage[sl].astype(BF16),
                preferred_element_type=F32,
            )
            out_ref[pl.ds(q, 1), :, jcols(j)] = (
                oblk.astype(BF16).reshape(1, S, SW)
            )
            r1, r2 = stripe_rdmas(j)
            r1.start()
            r2.start()
            if j > 0:
                p1, p2 = stripe_rdmas(j - 1)
                p1.wait_recv()
                p2.wait_recv()
                fy, fx = forward_rdmas(j - 1)
                fy.start()
                fx.start()

        p1, p2 = stripe_rdmas(NJ - 1)
        p1.wait_recv()
        p2.wait_recv()
        fy, fx = forward_rdmas(NJ - 1)
        fy.start()
        fx.start()
        for j in range(NJ):
            fy, fx = forward_rdmas(j)
            fy.wait_recv()
            fx.wait_recv()
        for j in range(NJ):
            r1, r2 = stripe_rdmas(j)
            r1.wait_send()
            r2.wait_send()
            fy, fx = forward_rdmas(j)
            fy.wait_send()
            fx.wait_send()


def kernel(x, Wdkv, Wuk, Wuv, Wq, Wqr, Wkr, Wo):
    return pl.pallas_call(
        _main_body,
        grid=(2 * G,),
        out_shape=jax.ShapeDtypeStruct((B, S, D), BF16),
        in_specs=[
            pl.BlockSpec(memory_space=pl.ANY),
            pl.BlockSpec(memory_space=pltpu.VMEM),
            pl.BlockSpec(memory_space=pltpu.VMEM),
            pl.BlockSpec(memory_space=pltpu.VMEM),
            pl.BlockSpec(memory_space=pltpu.VMEM),
            pl.BlockSpec(memory_space=pl.ANY),
            pl.BlockSpec((D, HPB * Dh),
                         lambda t: (0, jnp.minimum(t, G - 1))),
            pl.BlockSpec((D, HPB * Dr),
                         lambda t: (0, jnp.minimum(t, G - 1))),
        ],
        out_specs=pl.BlockSpec(memory_space=pltpu.VMEM),
        scratch_shapes=[
            pltpu.VMEM((S, D), F32),
            pltpu.VMEM((S, D), BF16),
            pltpu.VMEM((S, DCS), BF16),
            pltpu.VMEM((S, DCS), BF16),
            pltpu.VMEM((DCS, H * Dh), BF16),
            pltpu.VMEM((DCS, H * Dh), BF16),
            pltpu.VMEM((DCS, H * Dh), BF16),
            pltpu.VMEM((DCS, H * Dh), BF16),
            pltpu.VMEM((S, H * Dh), BF16),
            pltpu.VMEM((S, H * Dh), BF16),
            pltpu.VMEM((S, Dr), BF16),
            pltpu.VMEM((S, H * Dh), BF16),
            pltpu.VMEM((S, H * Dr), BF16),
            pltpu.VMEM((S, H * Dh), BF16),
            pltpu.VMEM((2, D, SW), F32),
            pltpu.SemaphoreType.DMA,
            pltpu.SemaphoreType.DMA((2,)),
            pltpu.SemaphoreType.DMA((3,)),
            pltpu.SemaphoreType.DMA((3,)),
            pltpu.SemaphoreType.DMA((4, NJ)),
            pltpu.SemaphoreType.DMA((4, NJ)),
        ],
        compiler_params=pltpu.CompilerParams(
            dimension_semantics=("arbitrary",),
            collective_id=0,
            vmem_limit_bytes=62 * 1024 * 1024,
        ),
    )(x, Wdkv, Wuk, Wuv, Wkr, Wo, Wq, Wqr)


# device time: 105630 ns/iter; 2.9388x vs baseline; 1.2152x over previous
import jax
import jax.numpy as jnp
from jax import lax
from jax.experimental import pallas as pl
from jax.experimental.pallas import tpu as pltpu

B, S, H, Dh, Dr = 4, 256, 32, 128, 64
D = 4096
DCS = 128
HD = H * Dh
HD2 = HD // 2
SCALE = (Dh + Dr) ** -0.5
BF16 = jnp.bfloat16
F32 = jnp.float32
GPZ = 8
GW = HD2 // GPZ
GRW = GW // Dh * Dr
NJ = 16
SW = D // NJ
SW2 = SW // 2

_MESH = pl.DeviceIdType.MESH


def _main_body(x_hbm, wdkv_ref, wuk_ref, wuv_ref, wkr_ref,
               wq_hbm, wqr_hbm, wo_hbm, out_ref,
               xq32, xb, c_loc, c_rem, wuk_loc, wuk_rem, wuv_loc, wuv_rem,
               kbuf, vbuf, kr_scr, qall, qrall, o_all,
               wq_stage, wqr_stage, wo_stage,
               copy_sem, wq_sems, wqr_sems, wo_sems,
               zsend, zrecv, osend, orecv, gsend, grecv):
    my_x = lax.axis_index("x")
    my_y = lax.axis_index("y")
    my_z = lax.axis_index("z")
    q = 2 * my_x + my_y
    q_xn = 2 * (1 - my_x) + my_y
    q_yn = 2 * my_x + (1 - my_y)
    zp = (my_x, my_y, 1 - my_z)
    xn = (1 - my_x, my_y, my_z)
    yn = (my_x, 1 - my_y, my_z)
    hbase = my_z * HD2
    rbase = my_z * (H * Dr // 2)

    def gq_cols(k):
        return pl.ds(hbase + k * GW, GW)

    def gq_rcols(k):
        return pl.ds(rbase + k * GRW, GRW)

    def grem_cols(k):
        return pl.ds((HD2 - hbase) + k * GW, GW)

    def wq_fetch(k, sl):
        pltpu.make_async_copy(
            wq_hbm.at[:, gq_cols(k)], wq_stage.at[sl], wq_sems.at[sl]
        ).start()
        pltpu.make_async_copy(
            wqr_hbm.at[:, gq_rcols(k)], wqr_stage.at[sl], wqr_sems.at[sl]
        ).start()

    def wq_wait(sl):
        pltpu.make_async_copy(
            wq_hbm.at[:, gq_cols(0)], wq_stage.at[sl], wq_sems.at[sl]
        ).wait()
        pltpu.make_async_copy(
            wqr_hbm.at[:, gq_rcols(0)], wqr_stage.at[sl], wqr_sems.at[sl]
        ).wait()

    def o_rdma(k, mine):
        cols = gq_cols(k) if mine else grem_cols(k)
        return pltpu.make_async_remote_copy(
            src_ref=o_all.at[:, cols], dst_ref=o_all.at[:, cols],
            send_sem=osend.at[k], recv_sem=orecv.at[k],
            device_id=zp, device_id_type=_MESH,
        )

    def z_rdmas():
        pairs = ((c_loc, c_rem), (wuk_loc, wuk_rem), (wuv_loc, wuv_rem))
        return [
            pltpu.make_async_remote_copy(
                src_ref=src, dst_ref=dst,
                send_sem=zsend.at[i], recv_sem=zrecv.at[i],
                device_id=zp, device_id_type=_MESH,
            )
            for i, (src, dst) in enumerate(pairs)
        ]

    def jcols(j, half=None):
        if half is None:
            return pl.ds(j * SW, SW)
        return pl.ds(j * SW + half * SW2, SW2)

    def stripe_rdmas(j):
        r1 = pltpu.make_async_remote_copy(
            src_ref=out_ref.at[q, :, jcols(j)],
            dst_ref=out_ref.at[q, :, jcols(j)],
            send_sem=gsend.at[0, j], recv_sem=grecv.at[0, j],
            device_id=xn, device_id_type=_MESH,
        )
        r2 = pltpu.make_async_remote_copy(
            src_ref=out_ref.at[q, :, jcols(j)],
            dst_ref=out_ref.at[q, :, jcols(j)],
            send_sem=gsend.at[1, j], recv_sem=grecv.at[1, j],
            device_id=yn, device_id_type=_MESH,
        )
        return r1, r2

    def forward_rdmas(j):
        fy = pltpu.make_async_remote_copy(
            src_ref=out_ref.at[q_xn, :, jcols(j, 0)],
            dst_ref=out_ref.at[q_xn, :, jcols(j, 0)],
            send_sem=gsend.at[2, j], recv_sem=grecv.at[2, j],
            device_id=yn, device_id_type=_MESH,
        )
        fx = pltpu.make_async_remote_copy(
            src_ref=out_ref.at[q_yn, :, jcols(j, 1)],
            dst_ref=out_ref.at[q_yn, :, jcols(j, 1)],
            send_sem=gsend.at[3, j], recv_sem=grecv.at[3, j],
            device_id=xn, device_id_type=_MESH,
        )
        return fy, fx

    wq_fetch(0, 0)
    cp = pltpu.make_async_copy(x_hbm.at[q], xq32, copy_sem)
    cp.start()
    cp.wait()
    xb[...] = xq32[...].astype(BF16)
    c_loc[...] = jnp.dot(
        xb[...], wdkv_ref[...].astype(BF16), preferred_element_type=F32
    ).astype(BF16)
    their_cols = pl.ds((HD2 - hbase), HD2)
    my_cols = pl.ds(hbase, HD2)
    wuk_loc[...] = wuk_ref[:, their_cols].astype(BF16)
    wuv_loc[...] = wuv_ref[:, their_cols].astype(BF16)

    barrier = pltpu.get_barrier_semaphore()
    for nbr in (zp, xn, yn):
        pl.semaphore_signal(barrier, inc=1, device_id=nbr,
                            device_id_type=_MESH)
    pl.semaphore_wait(barrier, 3)

    zr = z_rdmas()
    for r in zr:
        r.start()

    kr_scr[...] = jnp.dot(
        xb[...], wkr_ref[...].astype(BF16), preferred_element_type=F32
    ).astype(BF16)

    for k in range(GPZ):
        sl = k % 2
        wq_wait(sl)
        if k + 1 < GPZ:
            wq_fetch(k + 1, 1 - sl)
        qall[:, pl.ds(k * GW, GW)] = jnp.dot(
            xb[...], wq_stage[sl].astype(BF16), preferred_element_type=F32
        ).astype(BF16)
        qrall[:, pl.ds(k * GRW, GRW)] = jnp.dot(
            xb[...], wqr_stage[sl].astype(BF16), preferred_element_type=F32
        ).astype(BF16)

    for r in zr:
        r.wait()
    k_ = jnp.dot(c_loc[...], wuk_ref[:, my_cols].astype(BF16),
                 preferred_element_type=F32)
    k_ += jnp.dot(c_rem[...], wuk_rem[...], preferred_element_type=F32)
    kbuf[...] = k_.astype(BF16)
    v_ = jnp.dot(c_loc[...], wuv_ref[:, my_cols].astype(BF16),
                 preferred_element_type=F32)
    v_ += jnp.dot(c_rem[...], wuv_rem[...], preferred_element_type=F32)
    vbuf[...] = v_.astype(BF16)

    kr_t = kr_scr[...].T
    for k in range(GPZ):
        qr_g = qrall[:, pl.ds(k * GRW, GRW)]
        outs = []
        for hh in range(GW // Dh):
            col = pl.ds(k * GW + hh * Dh, Dh)
            q_h = qall[:, col]
            qr_h = qr_g[:, hh * Dr:(hh + 1) * Dr]
            k_h = kbuf[:, col]
            v_h = vbuf[:, col]
            s = jnp.dot(q_h, k_h.T, preferred_element_type=F32)
            s += jnp.dot(qr_h, kr_t, preferred_element_type=F32)
            s *= SCALE
            m = jnp.max(s, axis=-1, keepdims=True)
            p = jnp.exp(s - m)
            p = p / jnp.sum(p, axis=-1, keepdims=True)
            outs.append(
                jnp.dot(p.astype(BF16), v_h, preferred_element_type=F32)
            )
        o_all[:, gq_cols(k)] = jnp.concatenate(outs, axis=-1).astype(BF16)
        o_rdma(k, mine=True).start()

    ld0 = pltpu.make_async_copy(
        wo_hbm.at[:, jcols(0)], wo_stage.at[0], wo_sems.at[0]
    )
    ld0.start()
    for k in range(GPZ):
        o_rdma(k, mine=False).wait_recv()

    for j in range(NJ):
        sl = j % 2
        pltpu.make_async_copy(
            wo_hbm.at[:, jcols(j)], wo_stage.at[sl], wo_sems.at[sl]
        ).wait()
        if j + 1 < NJ:
            pltpu.make_async_copy(
                wo_hbm.at[:, jcols(j + 1)], wo_stage.at[1 - sl],
                wo_sems.at[1 - sl],
            ).start()
        oblk = jnp.dot(
            o_all[...], wo_stage[sl].astype(BF16),
            preferred_element_type=F32,
        )
        out_ref[pl.ds(q, 1), :, jcols(j)] = (
            oblk.astype(BF16).reshape(1, S, SW)
        )
        r1, r2 = stripe_rdmas(j)
        r1.start()
        r2.start()
        if j > 0:
            p1, p2 = stripe_rdmas(j - 1)
            p1.wait_recv()
            p2.wait_recv()
            fy, fx = forward_rdmas(j - 1)
            fy.start()
            fx.start()

    p1, p2 = stripe_rdmas(NJ - 1)
    p1.wait_recv()
    p2.wait_recv()
    fy, fx = forward_rdmas(NJ - 1)
    fy.start()
    fx.start()
    for j in range(NJ):
        fy, fx = forward_rdmas(j)
        fy.wait_recv()
        fx.wait_recv()
    for j in range(NJ):
        r1, r2 = stripe_rdmas(j)
        r1.wait_send()
        r2.wait_send()
        fy, fx = forward_rdmas(j)
        fy.wait_send()
        fx.wait_send()
    for k in range(GPZ):
        o_rdma(k, mine=True).wait_send()


def kernel(x, Wdkv, Wuk, Wuv, Wq, Wqr, Wkr, Wo):
    return pl.pallas_call(
        _main_body,
        out_shape=jax.ShapeDtypeStruct((B, S, D), BF16),
        in_specs=[
            pl.BlockSpec(memory_space=pl.ANY),
            pl.BlockSpec(memory_space=pltpu.VMEM),
            pl.BlockSpec(memory_space=pltpu.VMEM),
            pl.BlockSpec(memory_space=pltpu.VMEM),
            pl.BlockSpec(memory_space=pltpu.VMEM),
            pl.BlockSpec(memory_space=pl.ANY),
            pl.BlockSpec(memory_space=pl.ANY),
            pl.BlockSpec(memory_space=pl.ANY),
        ],
        out_specs=pl.BlockSpec(memory_space=pltpu.VMEM),
        scratch_shapes=[
            pltpu.VMEM((S, D), F32),
            pltpu.VMEM((S, D), BF16),
            pltpu.VMEM((S, DCS), BF16),
            pltpu.VMEM((S, DCS), BF16),
            pltpu.VMEM((DCS, HD2), BF16),
            pltpu.VMEM((DCS, HD2), BF16),
            pltpu.VMEM((DCS, HD2), BF16),
            pltpu.VMEM((DCS, HD2), BF16),
            pltpu.VMEM((S, HD2), BF16),
            pltpu.VMEM((S, HD2), BF16),
            pltpu.VMEM((S, Dr), BF16),
            pltpu.VMEM((S, HD2), BF16),
            pltpu.VMEM((S, H * Dr // 2), BF16),
            pltpu.VMEM((S, HD), BF16),
            pltpu.VMEM((2, D, GW), F32),
            pltpu.VMEM((2, D, GRW), F32),
            pltpu.VMEM((2, D, SW), F32),
            pltpu.SemaphoreType.DMA,
            pltpu.SemaphoreType.DMA((2,)),
            pltpu.SemaphoreType.DMA((2,)),
            pltpu.SemaphoreType.DMA((2,)),
            pltpu.SemaphoreType.DMA((3,)),
            pltpu.SemaphoreType.DMA((3,)),
            pltpu.SemaphoreType.DMA((GPZ,)),
            pltpu.SemaphoreType.DMA((GPZ,)),
            pltpu.SemaphoreType.DMA((4, NJ)),
            pltpu.SemaphoreType.DMA((4, NJ)),
        ],
        compiler_params=pltpu.CompilerParams(
            collective_id=0,
            vmem_limit_bytes=62 * 1024 * 1024,
        ),
    )(x, Wdkv, Wuk, Wuv, Wkr, Wq, Wqr, Wo)


# device time: 105154 ns/iter; 2.9521x vs baseline; 1.0045x over previous
import jax
import jax.numpy as jnp
from jax import lax
from jax.experimental import pallas as pl
from jax.experimental.pallas import tpu as pltpu

B, S, H, Dh, Dr = 4, 256, 32, 128, 64
D = 4096
DCS = 128
HD = H * Dh
HD2 = HD // 2
SCALE = (Dh + Dr) ** -0.5
BF16 = jnp.bfloat16
F32 = jnp.float32
GPZ = 8
GW = HD2 // GPZ
GRW = GW // Dh * Dr
NJ = 16
SW = D // NJ
SW2 = SW // 2

_MESH = pl.DeviceIdType.MESH


def _main_body(x_hbm, wdkv_ref, wuk_ref, wuv_ref, wkr_ref,
               wq_hbm, wqr_hbm, wo_hbm, out_ref,
               xq32, xb, c_loc, c_rem, wuk_loc, wuk_rem, wuv_loc, wuv_rem,
               kbuf, vbuf, kr_scr, qall, qrall, o_all,
               wq_stage, wqr_stage, wo_stage,
               copy_sem, wq_sems, wqr_sems, wo_sems,
               zsend, zrecv, osend, orecv, gsend, grecv):
    my_x = lax.axis_index("x")
    my_y = lax.axis_index("y")
    my_z = lax.axis_index("z")
    q = 2 * my_x + my_y
    q_xn = 2 * (1 - my_x) + my_y
    q_yn = 2 * my_x + (1 - my_y)
    zp = (my_x, my_y, 1 - my_z)
    xn = (1 - my_x, my_y, my_z)
    yn = (my_x, 1 - my_y, my_z)
    hbase = my_z * HD2
    rbase = my_z * (H * Dr // 2)

    def gq_cols(k):
        return pl.ds(hbase + k * GW, GW)

    def gq_rcols(k):
        return pl.ds(rbase + k * GRW, GRW)

    def grem_cols(k):
        return pl.ds((HD2 - hbase) + k * GW, GW)

    def wq_fetch(k, sl):
        pltpu.make_async_copy(
            wq_hbm.at[:, gq_cols(k)], wq_stage.at[sl], wq_sems.at[sl]
        ).start()
        pltpu.make_async_copy(
            wqr_hbm.at[:, gq_rcols(k)], wqr_stage.at[sl], wqr_sems.at[sl]
        ).start()

    def wq_wait(sl):
        pltpu.make_async_copy(
            wq_hbm.at[:, gq_cols(0)], wq_stage.at[sl], wq_sems.at[sl]
        ).wait()
        pltpu.make_async_copy(
            wqr_hbm.at[:, gq_rcols(0)], wqr_stage.at[sl], wqr_sems.at[sl]
        ).wait()

    def o_rdma(k, mine):
        cols = gq_cols(k) if mine else grem_cols(k)
        return pltpu.make_async_remote_copy(
            src_ref=o_all.at[:, cols], dst_ref=o_all.at[:, cols],
            send_sem=osend.at[k], recv_sem=orecv.at[k],
            device_id=zp, device_id_type=_MESH,
        )

    def z_rdmas():
        pairs = ((c_loc, c_rem), (wuk_loc, wuk_rem), (wuv_loc, wuv_rem))
        return [
            pltpu.make_async_remote_copy(
                src_ref=src, dst_ref=dst,
                send_sem=zsend.at[i], recv_sem=zrecv.at[i],
                device_id=zp, device_id_type=_MESH,
            )
            for i, (src, dst) in enumerate(pairs)
        ]

    def jcols(j, half=None):
        if half is None:
            return pl.ds(j * SW, SW)
        return pl.ds(j * SW + half * SW2, SW2)

    def stripe_rdmas(j):
        r1 = pltpu.make_async_remote_copy(
            src_ref=out_ref.at[q, :, jcols(j)],
            dst_ref=out_ref.at[q, :, jcols(j)],
            send_sem=gsend.at[0, j], recv_sem=grecv.at[0, j],
            device_id=xn, device_id_type=_MESH,
        )
        r2 = pltpu.make_async_remote_copy(
            src_ref=out_ref.at[q, :, jcols(j)],
            dst_ref=out_ref.at[q, :, jcols(j)],
            send_sem=gsend.at[1, j], recv_sem=grecv.at[1, j],
            device_id=yn, device_id_type=_MESH,
        )
        return r1, r2

    def forward_rdmas(j):
        fy = pltpu.make_async_remote_copy(
            src_ref=out_ref.at[q_xn, :, jcols(j, 0)],
            dst_ref=out_ref.at[q_xn, :, jcols(j, 0)],
            send_sem=gsend.at[2, j], recv_sem=grecv.at[2, j],
            device_id=yn, device_id_type=_MESH,
        )
        fx = pltpu.make_async_remote_copy(
            src_ref=out_ref.at[q_yn, :, jcols(j, 1)],
            dst_ref=out_ref.at[q_yn, :, jcols(j, 1)],
            send_sem=gsend.at[3, j], recv_sem=grecv.at[3, j],
            device_id=xn, device_id_type=_MESH,
        )
        return fy, fx

    wq_fetch(0, 0)
    cp = pltpu.make_async_copy(x_hbm.at[q], xq32, copy_sem)
    cp.start()
    cp.wait()
    xb[...] = xq32[...].astype(BF16)
    c_loc[...] = jnp.dot(
        xb[...], wdkv_ref[...].astype(BF16), preferred_element_type=F32
    ).astype(BF16)
    their_cols = pl.ds((HD2 - hbase), HD2)
    my_cols = pl.ds(hbase, HD2)
    wuk_loc[...] = wuk_ref[:, their_cols].astype(BF16)
    wuv_loc[...] = wuv_ref[:, their_cols].astype(BF16)

    barrier = pltpu.get_barrier_semaphore()
    for nbr in (zp, xn, yn):
        pl.semaphore_signal(barrier, inc=1, device_id=nbr,
                            device_id_type=_MESH)
    pl.semaphore_wait(barrier, 3)

    zr = z_rdmas()
    for r in zr:
        r.start()

    kr_scr[...] = jnp.dot(
        xb[...], wkr_ref[...].astype(BF16), preferred_element_type=F32
    ).astype(BF16)

    for k in range(GPZ):
        sl = k % 2
        wq_wait(sl)
        if k + 1 < GPZ:
            wq_fetch(k + 1, 1 - sl)
        qall[:, pl.ds(k * GW, GW)] = jnp.dot(
            xb[...], wq_stage[sl].astype(BF16), preferred_element_type=F32
        ).astype(BF16)
        qrall[:, pl.ds(k * GRW, GRW)] = jnp.dot(
            xb[...], wqr_stage[sl].astype(BF16), preferred_element_type=F32
        ).astype(BF16)

    for r in zr:
        r.wait()
    k_ = jnp.dot(c_loc[...], wuk_ref[:, my_cols].astype(BF16),
                 preferred_element_type=F32)
    k_ += jnp.dot(c_rem[...], wuk_rem[...], preferred_element_type=F32)
    kbuf[...] = k_.astype(BF16)
    v_ = jnp.dot(c_loc[...], wuv_ref[:, my_cols].astype(BF16),
                 preferred_element_type=F32)
    v_ += jnp.dot(c_rem[...], wuv_rem[...], preferred_element_type=F32)
    vbuf[...] = v_.astype(BF16)

    for sl in range(2):
        pltpu.make_async_copy(
            wo_hbm.at[:, jcols(sl)], wo_stage.at[sl], wo_sems.at[sl]
        ).start()

    kr_t = kr_scr[...].T
    for k in range(GPZ):
        qr_g = qrall[:, pl.ds(k * GRW, GRW)]
        outs = []
        for hh in range(GW // Dh):
            col = pl.ds(k * GW + hh * Dh, Dh)
            q_h = qall[:, col]
            qr_h = qr_g[:, hh * Dr:(hh + 1) * Dr]
            k_h = kbuf[:, col]
            v_h = vbuf[:, col]
            s = jnp.dot(q_h, k_h.T, preferred_element_type=F32)
            s += jnp.dot(qr_h, kr_t, preferred_element_type=F32)
            s *= SCALE
            m = jnp.max(s, axis=-1, keepdims=True)
            p = jnp.exp(s - m)
            p = p / jnp.sum(p, axis=-1, keepdims=True)
            outs.append(
                jnp.dot(p.astype(BF16), v_h, preferred_element_type=F32)
            )
        o_all[:, gq_cols(k)] = jnp.concatenate(outs, axis=-1).astype(BF16)
        o_rdma(k, mine=True).start()

    for k in range(GPZ):
        o_rdma(k, mine=False).wait_recv()

    for j in range(NJ):
        sl = j % 2
        pltpu.make_async_copy(
            wo_hbm.at[:, jcols(j)], wo_stage.at[sl], wo_sems.at[sl]
        ).wait()
        oblk = jnp.dot(
            o_all[...], wo_stage[sl].astype(BF16),
            preferred_element_type=F32,
        )
        if j + 2 < NJ:
            pltpu.make_async_copy(
                wo_hbm.at[:, jcols(j + 2)], wo_stage.at[sl],
                wo_sems.at[sl],
            ).start()
        out_ref[pl.ds(q, 1), :, jcols(j)] = (
            oblk.astype(BF16).reshape(1, S, SW)
        )
        r1, r2 = stripe_rdmas(j)
        r1.start()
        r2.start()
        if j > 0:
            p1, p2 = stripe_rdmas(j - 1)
            p1.wait_recv()
            p2.wait_recv()
            fy, fx = forward_rdmas(j - 1)
            fy.start()
            fx.start()

    p1, p2 = stripe_rdmas(NJ - 1)
    p1.wait_recv()
    p2.wait_recv()
    fy, fx = forward_rdmas(NJ - 1)
    fy.start()
    fx.start()
    for j in range(NJ):
        fy, fx = forward_rdmas(j)
        fy.wait_recv()
        fx.wait_recv()
    for j in range(NJ):
        r1, r2 = stripe_rdmas(j)
        r1.wait_send()
        r2.wait_send()
        fy, fx = forward_rdmas(j)
        fy.wait_send()
        fx.wait_send()
    for k in range(GPZ):
        o_rdma(k, mine=True).wait_send()


def kernel(x, Wdkv, Wuk, Wuv, Wq, Wqr, Wkr, Wo):
    return pl.pallas_call(
        _main_body,
        out_shape=jax.ShapeDtypeStruct((B, S, D), BF16),
        in_specs=[
            pl.BlockSpec(memory_space=pl.ANY),
            pl.BlockSpec(memory_space=pltpu.VMEM),
            pl.BlockSpec(memory_space=pltpu.VMEM),
            pl.BlockSpec(memory_space=pltpu.VMEM),
            pl.BlockSpec(memory_space=pltpu.VMEM),
            pl.BlockSpec(memory_space=pl.ANY),
            pl.BlockSpec(memory_space=pl.ANY),
            pl.BlockSpec(memory_space=pl.ANY),
        ],
        out_specs=pl.BlockSpec(memory_space=pltpu.VMEM),
        scratch_shapes=[
            pltpu.VMEM((S, D), F32),
            pltpu.VMEM((S, D), BF16),
            pltpu.VMEM((S, DCS), BF16),
            pltpu.VMEM((S, DCS), BF16),
            pltpu.VMEM((DCS, HD2), BF16),
            pltpu.VMEM((DCS, HD2), BF16),
            pltpu.VMEM((DCS, HD2), BF16),
            pltpu.VMEM((DCS, HD2), BF16),
            pltpu.VMEM((S, HD2), BF16),
            pltpu.VMEM((S, HD2), BF16),
            pltpu.VMEM((S, Dr), BF16),
            pltpu.VMEM((S, HD2), BF16),
            pltpu.VMEM((S, H * Dr // 2), BF16),
            pltpu.VMEM((S, HD), BF16),
            pltpu.VMEM((2, D, GW), F32),
            pltpu.VMEM((2, D, GRW), F32),
            pltpu.VMEM((2, D, SW), F32),
            pltpu.SemaphoreType.DMA,
            pltpu.SemaphoreType.DMA((2,)),
            pltpu.SemaphoreType.DMA((2,)),
            pltpu.SemaphoreType.DMA((2,)),
            pltpu.SemaphoreType.DMA((3,)),
            pltpu.SemaphoreType.DMA((3,)),
            pltpu.SemaphoreType.DMA((GPZ,)),
            pltpu.SemaphoreType.DMA((GPZ,)),
            pltpu.SemaphoreType.DMA((4, NJ)),
            pltpu.SemaphoreType.DMA((4, NJ)),
        ],
        compiler_params=pltpu.CompilerParams(
            collective_id=0,
            vmem_limit_bytes=62 * 1024 * 1024,
        ),
    )(x, Wdkv, Wuk, Wuv, Wkr, Wq, Wqr, Wo)
